# baseline (speedup 1.0000x reference)
import functools

import jax
import jax.numpy as jnp
from jax import lax
from jax.experimental import pallas as pl
from jax.experimental.pallas import tpu as pltpu
from jax.experimental.pallas import tpu_sc as plsc

_DIM = 64
_LANES = 128
_N_WORKERS = 32
_OROWS = _LANES // 2


def _gather_kernel(ids_hbm, wp_hbm, out_hbm, idx_v, idxp_v, gbuf, obuf,
                   gsem0, gsem1, wsem0, wsem1, *, rows_per_w):
    wid = lax.axis_index("s") * 2 + lax.axis_index("c")
    out_row0 = wid * rows_per_w * _OROWS

    pltpu.sync_copy(ids_hbm.at[wid], idx_v)

    def idx_body(j, carry):
        for g in range(_LANES // 16):
            sl = pl.ds(g * 16, 16)
            idxp_v[j, sl] = lax.shift_right_logical(idx_v[j, sl], 1)
        return carry
    lax.fori_loop(0, rows_per_w, idx_body, 0)

    gsems = (gsem0, gsem1)
    wsems = (wsem0, wsem1)

    def gather_start(j, h):
        pltpu.make_async_copy(wp_hbm.at[idxp_v.at[j]], gbuf.at[h],
                              gsems[h]).start()

    def gather_wait(h):
        pltpu.make_async_copy(wp_hbm.at[idxp_v.at[0]], gbuf.at[h],
                              gsems[h]).wait()

    def wb_start(j, h):
        pltpu.make_async_copy(obuf.at[h],
                              out_hbm.at[pl.ds(out_row0 + j * _OROWS, _OROWS)],
                              wsems[h]).start()

    def wb_wait(j, h):
        pltpu.make_async_copy(obuf.at[h],
                              out_hbm.at[pl.ds(out_row0 + j * _OROWS, _OROWS)],
                              wsems[h]).wait()

    def merge(j, h):
        def mbody(g, carry):
            hv = (idx_v[j, pl.ds(g * 16, 16)] & 1) * _DIM
            for l in range(16):
                t = g * 16 + l
                i = g * 8 + l // 2
                c0 = (l % 2) * _DIM
                ho = hv[l]
                for k in range(_DIM // 16):
                    obuf[h, i, pl.ds(c0 + k * 16, 16)] = \
                        gbuf[h, t, pl.ds(ho + k * 16, 16)]
            return carry
        lax.fori_loop(0, _LANES // 16, mbody, 0)

    gather_start(0, 0)

    def block(s, carry):
        j0 = 2 * s
        j1 = 2 * s + 1
        gather_start(j1, 1)
        gather_wait(0)

        @pl.when(s >= 1)
        def _():
            wb_wait(j0 - 2, 0)
        merge(j0, 0)
        wb_start(j0, 0)

        @pl.when(s < rows_per_w // 2 - 1)
        def _():
            gather_start(j1 + 1, 0)
        gather_wait(1)

        @pl.when(s >= 1)
        def _():
            wb_wait(j1 - 2, 1)
        merge(j1, 1)
        wb_start(j1, 1)
        return carry

    lax.fori_loop(0, rows_per_w // 2, block, 0)
    wb_wait(rows_per_w - 2, 0)
    wb_wait(rows_per_w - 1, 1)


def kernel(input_ids, weight):
    b, s = input_ids.shape
    n, d = weight.shape
    total = b * s
    n_rows = total // _LANES
    rows_per_w = n_rows // _N_WORKERS

    ids3d = input_ids.reshape(_N_WORKERS, rows_per_w, _LANES).astype(jnp.int32)
    wpair = weight.reshape(n // 2, 2 * d)
    mesh = plsc.VectorSubcoreMesh(core_axis_name="c", subcore_axis_name="s")

    run = functools.partial(
        pl.kernel,
        mesh=mesh,
        out_type=jax.ShapeDtypeStruct((total // 2, 2 * d), jnp.float32),
        scratch_types=[
            pltpu.VMEM((rows_per_w, _LANES), jnp.int32),
            pltpu.VMEM((rows_per_w, _LANES), jnp.int32),
            pltpu.VMEM((2, _LANES, 2 * d), jnp.float32),
            pltpu.VMEM((2, _OROWS, 2 * d), jnp.float32),
            pltpu.SemaphoreType.DMA,
            pltpu.SemaphoreType.DMA,
            pltpu.SemaphoreType.DMA,
            pltpu.SemaphoreType.DMA,
        ],
        compiler_params=pltpu.CompilerParams(use_tc_tiling_on_sc=True),
    )(functools.partial(_gather_kernel, rows_per_w=rows_per_w))

    out = run(ids3d, wpair)
    return out.reshape(b, s, d)
